# baseline (speedup 1.0000x reference)
import jax
import jax.numpy as jnp
from jax.experimental import pallas as pl
from jax.experimental.pallas import tpu as pltpu

B, H, S, D, Q = 16, 16, 2048, 128, 16
BH = B * H
CB = 4
NCHUNK = BH // CB
NITEM = 2 * NCHUNK
NBUF = 8


def _body(pos_ref, kc_ref, vc_ref, kv_ref, vv_ref, ko_ref, vo_ref,
          buf, insem, outsem):
    def item(j):
        c = j % 2
        i = j // 2
        src = kc_ref if c == 0 else vc_ref
        dst = ko_ref if c == 0 else vo_ref
        val = kv_ref if c == 0 else vv_ref
        return src, dst, val, i

    def start_in(j):
        b = j % NBUF
        src, dst, val, i = item(j)
        pltpu.make_async_copy(src.at[pl.ds(i * CB, CB)], buf.at[b],
                              insem.at[b]).start()

    def wait_in(j):
        b = j % NBUF
        src, dst, val, i = item(j)
        pltpu.make_async_copy(src.at[pl.ds(i * CB, CB)], buf.at[b],
                              insem.at[b]).wait()

    def start_out(j):
        b = j % NBUF
        src, dst, val, i = item(j)
        pltpu.make_async_copy(buf.at[b], dst.at[pl.ds(i * CB, CB)],
                              outsem.at[b]).start()

    def wait_out(j):
        b = j % NBUF
        src, dst, val, i = item(j)
        pltpu.make_async_copy(buf.at[b], dst.at[pl.ds(i * CB, CB)],
                              outsem.at[b]).wait()

    def scatter(j):
        b = j % NBUF
        src, dst, val, i = item(j)
        for c in range(CB):
            for q in range(Q):
                p = pos_ref[q]
                buf[b, c, pl.ds(p, 1), :] = val[i * CB + c, pl.ds(q, 1), :]

    for j in range(NBUF):
        start_in(j)
    for j in range(NITEM):
        wait_in(j)
        scatter(j)
        start_out(j)
        jn = j + NBUF
        if jn < NITEM:
            wait_out(j)
            start_in(jn)
    for j in range(NITEM - NBUF, NITEM):
        wait_out(j)


def kernel(k_cache, v_cache, input_pos, k_val, v_val):
    kc = k_cache.reshape(BH, S, D)
    vc = v_cache.reshape(BH, S, D)
    kv = k_val.reshape(BH, Q, D)
    vv = v_val.reshape(BH, Q, D)

    grid_spec = pltpu.PrefetchScalarGridSpec(
        num_scalar_prefetch=1,
        grid=(1,),
        in_specs=[
            pl.BlockSpec(memory_space=pltpu.MemorySpace.HBM),
            pl.BlockSpec(memory_space=pltpu.MemorySpace.HBM),
            pl.BlockSpec((BH, Q, D), lambda i, pos: (0, 0, 0)),
            pl.BlockSpec((BH, Q, D), lambda i, pos: (0, 0, 0)),
        ],
        out_specs=[
            pl.BlockSpec(memory_space=pltpu.MemorySpace.HBM),
            pl.BlockSpec(memory_space=pltpu.MemorySpace.HBM),
        ],
        scratch_shapes=[
            pltpu.VMEM((NBUF, CB, S, D), jnp.float32),
            pltpu.SemaphoreType.DMA((NBUF,)),
            pltpu.SemaphoreType.DMA((NBUF,)),
        ],
    )

    k_out, v_out = pl.pallas_call(
        _body,
        grid_spec=grid_spec,
        out_shape=[
            jax.ShapeDtypeStruct((BH, S, D), jnp.float32),
            jax.ShapeDtypeStruct((BH, S, D), jnp.float32),
        ],
    )(input_pos, kc, vc, kv, vv)

    return (k_out.reshape(B, H, S, D), v_out.reshape(B, H, S, D))


# speedup vs baseline: 1.0461x; 1.0461x over previous
import jax
import jax.numpy as jnp
from jax.experimental import pallas as pl
from jax.experimental.pallas import tpu as pltpu

B, H, S, D, Q = 16, 16, 2048, 128, 16
BH = B * H
CB = 4


def _body(pos_ref, kc_ref, vc_ref, kv_ref, vv_ref, ko_ref, vo_ref):
    ko_ref[...] = kc_ref[...]
    vo_ref[...] = vc_ref[...]
    for c in range(CB):
        for q in range(Q):
            p = pos_ref[q]
            ko_ref[c, pl.ds(p, 1), :] = kv_ref[c, pl.ds(q, 1), :]
            vo_ref[c, pl.ds(p, 1), :] = vv_ref[c, pl.ds(q, 1), :]


def kernel(k_cache, v_cache, input_pos, k_val, v_val):
    kc = k_cache.reshape(BH, S, D)
    vc = v_cache.reshape(BH, S, D)
    kv = k_val.reshape(BH, Q, D)
    vv = v_val.reshape(BH, Q, D)

    grid_spec = pltpu.PrefetchScalarGridSpec(
        num_scalar_prefetch=1,
        grid=(BH // CB,),
        in_specs=[
            pl.BlockSpec((CB, S, D), lambda i, pos: (i, 0, 0)),
            pl.BlockSpec((CB, S, D), lambda i, pos: (i, 0, 0)),
            pl.BlockSpec((CB, Q, D), lambda i, pos: (i, 0, 0)),
            pl.BlockSpec((CB, Q, D), lambda i, pos: (i, 0, 0)),
        ],
        out_specs=[
            pl.BlockSpec((CB, S, D), lambda i, pos: (i, 0, 0)),
            pl.BlockSpec((CB, S, D), lambda i, pos: (i, 0, 0)),
        ],
    )

    k_out, v_out = pl.pallas_call(
        _body,
        grid_spec=grid_spec,
        out_shape=[
            jax.ShapeDtypeStruct((BH, S, D), jnp.float32),
            jax.ShapeDtypeStruct((BH, S, D), jnp.float32),
        ],
        compiler_params=pltpu.CompilerParams(
            dimension_semantics=("arbitrary",),
        ),
    )(input_pos, kc, vc, kv, vv)

    return (k_out.reshape(B, H, S, D), v_out.reshape(B, H, S, D))


# speedup vs baseline: 1.0473x; 1.0011x over previous
import jax
import jax.numpy as jnp
from jax.experimental import pallas as pl
from jax.experimental.pallas import tpu as pltpu

B, H, S, D, Q = 16, 16, 2048, 128, 16
BH = B * H
CB = 4


def _body(pos_ref, kc_ref, vc_ref, kv_ref, vv_ref, ko_ref, vo_ref):
    ko_ref[...] = kc_ref[...]
    vo_ref[...] = vc_ref[...]
    for c in range(CB):
        for q in range(Q):
            p = pos_ref[q]
            ko_ref[c, pl.ds(p, 1), :] = kv_ref[c, pl.ds(q, 1), :]
            vo_ref[c, pl.ds(p, 1), :] = vv_ref[c, pl.ds(q, 1), :]


def kernel(k_cache, v_cache, input_pos, k_val, v_val):
    kc = k_cache.reshape(BH, S, D)
    vc = v_cache.reshape(BH, S, D)
    kv = k_val.reshape(BH, Q, D)
    vv = v_val.reshape(BH, Q, D)

    grid_spec = pltpu.PrefetchScalarGridSpec(
        num_scalar_prefetch=1,
        grid=(BH // CB,),
        in_specs=[
            pl.BlockSpec((CB, S, D), lambda i, pos: (i, 0, 0)),
            pl.BlockSpec((CB, S, D), lambda i, pos: (i, 0, 0)),
            pl.BlockSpec((CB, Q, D), lambda i, pos: (i, 0, 0)),
            pl.BlockSpec((CB, Q, D), lambda i, pos: (i, 0, 0)),
        ],
        out_specs=[
            pl.BlockSpec((CB, S, D), lambda i, pos: (i, 0, 0)),
            pl.BlockSpec((CB, S, D), lambda i, pos: (i, 0, 0)),
        ],
    )

    k_out, v_out = pl.pallas_call(
        _body,
        grid_spec=grid_spec,
        out_shape=[
            jax.ShapeDtypeStruct((BH, S, D), jnp.float32),
            jax.ShapeDtypeStruct((BH, S, D), jnp.float32),
        ],
        compiler_params=pltpu.CompilerParams(
            dimension_semantics=("parallel",),
        ),
    )(input_pos, kc, vc, kv, vv)

    return (k_out.reshape(B, H, S, D), v_out.reshape(B, H, S, D))
